# Initial kernel scaffold; baseline (speedup 1.0000x reference)
import jax
import jax.numpy as jnp
from jax.experimental import pallas as pl


def kernel(x, table, W1, b1, W2, b2, W3, b3):
    raise NotImplementedError("write your pallas kernel here")


# speedup vs baseline: 10.8066x; 10.8066x over previous
import functools

import jax
import jax.numpy as jnp
from jax import lax
from jax.experimental import pallas as pl
from jax.experimental.pallas import tpu as pltpu
from jax.experimental.pallas import tpu_sc as plsc

B = 16384
L = 200
EMB = 32
NUM_CLASSES = 10

NC = 2
NS = 16
NW = NC * NS
BPW = B // NW
RPW = BPW * L
GCH = 128
CHUNK = 1024
SUB = CHUNK // GCH
NCH = RPW // CHUNK
LANES = 16


def _pool_sc(x_flat, table, seg, zrows):
    mesh = plsc.VectorSubcoreMesh(core_axis_name="c", subcore_axis_name="s")

    @functools.partial(
        pl.kernel,
        out_type=jax.ShapeDtypeStruct((B, EMB), jnp.float32),
        mesh=mesh,
        scratch_types=[
            pltpu.VMEM((CHUNK,), jnp.int32),
            pltpu.VMEM((SUB, GCH), jnp.int32),
            pltpu.VMEM((SUB, GCH, EMB), jnp.float32),
            pltpu.VMEM_SHARED((NS * BPW, EMB), jnp.float32),
            pltpu.SemaphoreType.DMA((SUB,)),
        ],
        compiler_params=pltpu.CompilerParams(use_tc_tiling_on_sc=False),
    )
    def k(x_hbm, tab_hbm, seg_hbm, z_hbm, out_hbm, idx_v, seg_v, rows_v, acc_sh, gsem):
        sid = lax.axis_index("s")
        wid = lax.axis_index("c") * NS + sid
        base = wid * RPW

        pltpu.sync_copy(z_hbm, acc_sh.at[pl.ds(sid * BPW, BPW)])

        @pl.loop(0, NCH)
        def _chunk(c):
            pltpu.sync_copy(x_hbm.at[pl.ds(base + c * CHUNK, CHUNK)], idx_v)
            pltpu.sync_copy(
                seg_hbm.at[pl.ds(sid * (NCH * SUB) + c * SUB, SUB)], seg_v)
            handles = [
                pltpu.async_copy(
                    tab_hbm.at[idx_v.at[pl.ds(g * GCH, GCH)]],
                    rows_v.at[g],
                    gsem.at[g],
                )
                for g in range(SUB)
            ]
            for g in range(SUB):
                handles[g].wait()
                pltpu.sync_copy(rows_v.at[g], acc_sh.at[seg_v.at[g]], add=True)

        pltpu.sync_copy(acc_sh.at[pl.ds(sid * BPW, BPW)],
                        out_hbm.at[pl.ds(wid * BPW, BPW)])

    return k(x_flat, table, seg, zrows)


def _mlp_tc(pooled_sum, w1t, b1, w2t, b2, w3t, b3):
    def body(p_ref, w1_ref, b1_ref, w2_ref, b2_ref, w3_ref, b3_ref, o_ref):
        p = p_ref[...] * (1.0 / L)
        h = jnp.dot(p, w1_ref[...], precision=lax.Precision.HIGHEST,
                    preferred_element_type=jnp.float32) + b1_ref[...]
        h = jnp.maximum(h, 0.0)
        h = jnp.dot(h, w2_ref[...], precision=lax.Precision.HIGHEST,
                    preferred_element_type=jnp.float32) + b2_ref[...]
        h = jnp.maximum(h, 0.0)
        o_ref[...] = jnp.dot(h, w3_ref[...], precision=lax.Precision.HIGHEST,
                             preferred_element_type=jnp.float32) + b3_ref[...]

    return pl.pallas_call(
        body,
        out_shape=jax.ShapeDtypeStruct((B, NUM_CLASSES), jnp.float32),
    )(pooled_sum, w1t, b1, w2t, b2, w3t, b3)


def kernel(x, table, W1, b1, W2, b2, W3, b3):
    x_flat = x.reshape(-1)
    seg = (jnp.arange(NS * RPW, dtype=jnp.int32) // L).reshape(-1, GCH)
    zrows = jnp.zeros((BPW, EMB), dtype=jnp.float32)
    pooled_sum = _pool_sc(x_flat, table, seg, zrows)
    return _mlp_tc(
        pooled_sum,
        W1.T, b1.reshape(1, -1),
        W2.T, b2.reshape(1, -1),
        W3.T, b3.reshape(1, -1),
    )


# speedup vs baseline: 12.3724x; 1.1449x over previous
import functools

import jax
import jax.numpy as jnp
from jax import lax
from jax.experimental import pallas as pl
from jax.experimental.pallas import tpu as pltpu
from jax.experimental.pallas import tpu_sc as plsc

B = 16384
L = 200
EMB = 32
NUM_CLASSES = 10

NC = 2
NS = 16
NW = NC * NS
BPW = B // NW
RPW = BPW * L
GCH = 128
CHUNK = 1024
SUB = CHUNK // GCH
NCH = RPW // CHUNK
LANES = 16


def _pool_sc(x_flat, table, seg, zrows):
    mesh = plsc.VectorSubcoreMesh(core_axis_name="c", subcore_axis_name="s")

    @functools.partial(
        pl.kernel,
        out_type=jax.ShapeDtypeStruct((B, EMB), jnp.float32),
        mesh=mesh,
        scratch_types=[
            pltpu.VMEM((2, CHUNK), jnp.int32),
            pltpu.VMEM((2, SUB, GCH), jnp.int32),
            pltpu.VMEM((2, SUB, GCH, EMB), jnp.float32),
            pltpu.VMEM_SHARED((NS * BPW, EMB), jnp.float32),
            pltpu.SemaphoreType.DMA((2,)),
            pltpu.SemaphoreType.DMA((2,)),
            pltpu.SemaphoreType.DMA((2, SUB)),
        ],
        compiler_params=pltpu.CompilerParams(use_tc_tiling_on_sc=False),
    )
    def k(x_hbm, tab_hbm, seg_hbm, z_hbm, out_hbm,
          idx_v, seg_v, rows_v, acc_sh, isem, ssem, gsem):
        sid = lax.axis_index("s")
        wid = lax.axis_index("c") * NS + sid
        base = wid * RPW
        segbase = sid * (NCH * SUB)

        pltpu.sync_copy(z_hbm, acc_sh.at[pl.ds(sid * BPW, BPW)])

        def load_idx(c, b):
            pltpu.async_copy(x_hbm.at[pl.ds(base + c * CHUNK, CHUNK)],
                             idx_v.at[b], isem.at[b])
            pltpu.async_copy(seg_hbm.at[pl.ds(segbase + c * SUB, SUB)],
                             seg_v.at[b], ssem.at[b])

        def wait_idx(b):
            pltpu.make_async_copy(x_hbm.at[pl.ds(0, CHUNK)],
                                  idx_v.at[b], isem.at[b]).wait()
            pltpu.make_async_copy(seg_hbm.at[pl.ds(0, SUB)],
                                  seg_v.at[b], ssem.at[b]).wait()

        def fire_gathers(b):
            for g in range(SUB):
                pltpu.async_copy(
                    tab_hbm.at[idx_v.at[b, pl.ds(g * GCH, GCH)]],
                    rows_v.at[b, g],
                    gsem.at[b, g],
                )

        def wait_gathers(b):
            for g in range(SUB):
                pltpu.make_async_copy(tab_hbm.at[pl.ds(0, GCH)],
                                      rows_v.at[b, g], gsem.at[b, g]).wait()

        load_idx(0, 0)
        wait_idx(0)
        fire_gathers(0)
        load_idx(1, 1)

        @pl.loop(0, NCH, step=2)
        def _chunk(c0):
            for b in range(2):
                c = c0 + b
                nb = 1 - b

                @pl.when(c + 1 < NCH)
                def _():
                    wait_idx(nb)
                    fire_gathers(nb)

                wait_gathers(b)
                for g in range(SUB):
                    pltpu.sync_copy(rows_v.at[b, g], acc_sh.at[seg_v.at[b, g]],
                                    add=True)

                @pl.when(c + 2 < NCH)
                def _():
                    load_idx(c + 2, b)

        pltpu.sync_copy(acc_sh.at[pl.ds(sid * BPW, BPW)],
                        out_hbm.at[pl.ds(wid * BPW, BPW)])

    return k(x_flat, table, seg, zrows)


def _mlp_tc(pooled_sum, w1t, b1, w2t, b2, w3t, b3):
    def body(p_ref, w1_ref, b1_ref, w2_ref, b2_ref, w3_ref, b3_ref, o_ref):
        p = p_ref[...] * (1.0 / L)
        h = jnp.dot(p, w1_ref[...], precision=lax.Precision.HIGHEST,
                    preferred_element_type=jnp.float32) + b1_ref[...]
        h = jnp.maximum(h, 0.0)
        h = jnp.dot(h, w2_ref[...], precision=lax.Precision.HIGHEST,
                    preferred_element_type=jnp.float32) + b2_ref[...]
        h = jnp.maximum(h, 0.0)
        o_ref[...] = jnp.dot(h, w3_ref[...], precision=lax.Precision.HIGHEST,
                             preferred_element_type=jnp.float32) + b3_ref[...]

    return pl.pallas_call(
        body,
        out_shape=jax.ShapeDtypeStruct((B, NUM_CLASSES), jnp.float32),
    )(pooled_sum, w1t, b1, w2t, b2, w3t, b3)


def kernel(x, table, W1, b1, W2, b2, W3, b3):
    x_flat = x.reshape(-1)
    seg = (jnp.arange(NS * RPW, dtype=jnp.int32) // L).reshape(-1, GCH)
    zrows = jnp.zeros((BPW, EMB), dtype=jnp.float32)
    pooled_sum = _pool_sc(x_flat, table, seg, zrows)
    return _mlp_tc(
        pooled_sum,
        W1.T, b1.reshape(1, -1),
        W2.T, b2.reshape(1, -1),
        W3.T, b3.reshape(1, -1),
    )
